# baseline (speedup 1.0000x reference)
import functools

import jax
import jax.numpy as jnp
from jax import lax
from jax.experimental import pallas as pl
from jax.experimental.pallas import tpu as pltpu
from jax.experimental.pallas import tpu_sc as plsc

DIM = 32
B = 16384
MARGIN = 1.0
C = 0.25

NC = 2
NS = 16
NW = NC * NS

EB = 4 * B // NW
RB = 2 * B // NW


def _sc_gather(ent_emb, ent_idx, rel_emb, rel_idx):
    mesh = plsc.VectorSubcoreMesh(core_axis_name="c", subcore_axis_name="s")

    @functools.partial(
        pl.kernel,
        out_type=(
            jax.ShapeDtypeStruct((4 * B, DIM), jnp.float32),
            jax.ShapeDtypeStruct((2 * B, DIM), jnp.float32),
        ),
        mesh=mesh,
        scratch_types=[
            pltpu.VMEM((EB,), jnp.int32),
            pltpu.VMEM((EB, DIM), jnp.float32),
            pltpu.VMEM((RB,), jnp.int32),
            pltpu.VMEM((RB, DIM), jnp.float32),
            pltpu.SemaphoreType.DMA,
        ],
        compiler_params=pltpu.CompilerParams(use_tc_tiling_on_sc=False),
    )
    def k(ent_hbm, eidx_hbm, rel_hbm, ridx_hbm, ent_out, rel_out,
          eidx_v, erows_v, ridx_v, rrows_v, sem):
        wid = lax.axis_index("s") * NC + lax.axis_index("c")
        eb = wid * EB
        rb = wid * RB
        pltpu.sync_copy(eidx_hbm.at[pl.ds(eb, EB)], eidx_v)
        pltpu.sync_copy(ridx_hbm.at[pl.ds(rb, RB)], ridx_v)
        ec = pltpu.async_copy(ent_hbm.at[eidx_v], erows_v, sem)
        rc = pltpu.async_copy(rel_hbm.at[ridx_v], rrows_v, sem)
        ec.wait()
        rc.wait()
        pltpu.sync_copy(erows_v, ent_out.at[pl.ds(eb, EB)])
        pltpu.sync_copy(rrows_v, rel_out.at[pl.ds(rb, RB)])

    return k(ent_emb, ent_idx, rel_emb, rel_idx)


_TC_CHUNK = 2048


def _tc_body(h_ref, r_ref, t_ref, nh_ref, nr_ref, nt_ref, out_ref):
    h = h_ref[...]
    r = r_ref[...]
    t = t_ref[...]
    nh = nh_ref[...]
    nr = nr_ref[...]
    nt = nt_ref[...]

    pd = h + r - t
    nd = nh + nr - nt
    psq = jnp.sum(pd * pd, axis=1, keepdims=True)
    nsq = jnp.sum(nd * nd, axis=1, keepdims=True)
    marg = jnp.maximum(jnp.sqrt(psq) - jnp.sqrt(nsq) + MARGIN, 0.0)

    def rowreg(x):
        return jnp.maximum(jnp.sum(x * x, axis=1, keepdims=True) - 1.0, 0.0)

    ereg = rowreg(h) + rowreg(t) + rowreg(nh) + rowreg(nt)
    rreg = rowreg(r) + rowreg(nr)

    val = (jnp.sum(marg) / B
           + C * (jnp.sum(ereg) / (4 * B) + jnp.sum(rreg) / (2 * B)))

    @pl.when(pl.program_id(0) == 0)
    def _():
        out_ref[0, 0] = 0.0

    out_ref[0, 0] += val


def _tc_loss(ent_rows, rel_rows):
    grid = B // _TC_CHUNK
    blk = (_TC_CHUNK, DIM)

    def espec(region):
        return pl.BlockSpec(blk, lambda c, region=region: (region * grid + c, 0))

    out = pl.pallas_call(
        _tc_body,
        grid=(grid,),
        in_specs=[
            espec(0),
            pl.BlockSpec(blk, lambda c: (c, 0)),
            espec(1),
            espec(2),
            pl.BlockSpec(blk, lambda c: (grid + c, 0)),
            espec(3),
        ],
        out_specs=pl.BlockSpec(
            (1, 1), lambda c: (0, 0), memory_space=pltpu.SMEM),
        out_shape=jax.ShapeDtypeStruct((1, 1), jnp.float32),
    )(ent_rows, rel_rows, ent_rows, ent_rows, rel_rows, ent_rows)
    return out


def kernel(current_triples, corrupted_triples, ent_emb_1, rel_emb_1):
    ent_idx = jnp.concatenate([
        current_triples[:, 0], current_triples[:, 2],
        corrupted_triples[:, 0], corrupted_triples[:, 2],
    ])
    rel_idx = jnp.concatenate([current_triples[:, 1], corrupted_triples[:, 1]])
    ent_rows, rel_rows = _sc_gather(ent_emb_1, ent_idx, rel_emb_1, rel_idx)
    out = _tc_loss(ent_rows, rel_rows)
    return jnp.reshape(out, ())
